# baseline (speedup 1.0000x reference)
import jax
import jax.numpy as jnp
from jax.experimental import pallas as pl


def _phi_body(x_ref, y_ref, s_ref, t_ref, o_ref):
    th = t_ref[0, 0]
    x = x_ref[...]
    y = y_ref[...]
    ta = (0.5 * th) * (y - x)
    tb = (0.5 * th) * (y + x)
    ts = th * s_ref[...]
    w = jnp.abs(ts - tb) - ta
    o_ref[...] = 1.0 / (1.0 + jnp.exp(w))


def kernel(diagrams, samples, theta):
    n, p, _ = diagrams.shape
    s = samples.shape[0]
    rows = n * p
    blk = 2048

    xs = diagrams[..., 0].reshape(rows, 1)
    ys = diagrams[..., 1].reshape(rows, 1)
    s2 = samples.reshape(1, s)
    t2 = jnp.reshape(theta, (1, 1))

    out = pl.pallas_call(
        _phi_body,
        grid=(rows // blk,),
        in_specs=[
            pl.BlockSpec((blk, 1), lambda i: (i, 0)),
            pl.BlockSpec((blk, 1), lambda i: (i, 0)),
            pl.BlockSpec((1, s), lambda i: (0, 0)),
            pl.BlockSpec((1, 1), lambda i: (0, 0)),
        ],
        out_specs=pl.BlockSpec((blk, s), lambda i: (i, 0)),
        out_shape=jax.ShapeDtypeStruct((rows, s), jnp.float32),
    )(xs, ys, s2, t2)

    output = out.reshape(n, p, s)
    output_shape = jnp.array(samples.shape, dtype=jnp.int32)
    return (output, output_shape)
